# baseline (speedup 1.0000x reference)
import functools
import math

import jax
import jax.numpy as jnp
from jax import lax
from jax.experimental import pallas as pl
from jax.experimental.pallas import tpu as pltpu
from jax.experimental.pallas import tpu_sc as plsc

NUM_CORES = 2
NUM_SUBCORES = 16
NUM_WORKERS = NUM_CORES * NUM_SUBCORES

CHUNK = 16
NT = 3
NS = 3
LOOKAHEAD = 2


@functools.partial(jax.jit, static_argnames=())
def _gather_rows(flat_idx, table):
    n = flat_idx.shape[0]
    d = table.shape[1]
    n_per_w = n // NUM_WORKERS
    n_chunks = n_per_w // CHUNK

    mesh = plsc.VectorSubcoreMesh(core_axis_name="c", subcore_axis_name="s")
    G = LOOKAHEAD

    @functools.partial(
        pl.kernel,
        mesh=mesh,
        out_type=jax.ShapeDtypeStruct((n, d), jnp.float32),
        scratch_types=[
            pltpu.VMEM((n_per_w,), jnp.int32),
            *[pltpu.VMEM((CHUNK, d), jnp.float32) for _ in range(NT)],
            pltpu.VMEM_SHARED((NUM_SUBCORES, NS, CHUNK, d), jnp.float32),
            *[pltpu.SemaphoreType.DMA for _ in range(2 * NT + 2 * NS)],
        ],
    )
    def k(idx_hbm, table_hbm, out_hbm, idx_v, *rest):
        tbufs = rest[:NT]
        shared = rest[NT]
        sems = rest[NT + 1 :]
        gsems = sems[:NT]
        tsems = sems[NT : NT + NS]
        wsems = sems[NT + NS : NT + 2 * NS]
        dsems = sems[NT + 2 * NS :]

        sid = lax.axis_index("s")
        wid = sid * NUM_CORES + lax.axis_index("c")
        base = wid * n_per_w
        pltpu.sync_copy(idx_hbm.at[pl.ds(base, n_per_w)], idx_v)

        def gather(i, t, s=0):
            pltpu.async_copy(
                table_hbm.at[idx_v.at[pl.ds(i * CHUNK, CHUNK)]], tbufs[t], gsems[t]
            )

        def wait_gather(i, t, s=0):
            pltpu.make_async_copy(
                table_hbm.at[idx_v.at[pl.ds(i * CHUNK, CHUNK)]], tbufs[t], gsems[t]
            ).wait()

        def t2s(i, t, s):
            pltpu.async_copy(tbufs[t], shared.at[sid, s], tsems[s])

        def wait_t2s(i, t, s):
            pltpu.make_async_copy(tbufs[t], shared.at[sid, s], tsems[s]).wait()

        def write(i, t, s):
            pltpu.async_copy(
                shared.at[sid, s], out_hbm.at[pl.ds(base + i * CHUNK, CHUNK)], wsems[s]
            )

        def wait_write(i, t, s):
            pltpu.make_async_copy(
                shared.at[sid, s], out_hbm.at[pl.ds(base + i * CHUNK, CHUNK)], wsems[s]
            ).wait()

        def dwrite(i, t, s=0):
            pltpu.async_copy(
                tbufs[t], out_hbm.at[pl.ds(base + i * CHUNK, CHUNK)], dsems[t]
            )

        def wait_dwrite(i, t, s=0):
            pltpu.make_async_copy(
                tbufs[t], out_hbm.at[pl.ds(base + i * CHUNK, CHUNK)], dsems[t]
            ).wait()

        LCM = math.lcm(NT, NS)

        def sel(i, fn):
            for bb in range(LCM):

                @pl.when(lax.rem(i, LCM) == bb)
                def _():
                    fn(i, bb % NT, bb % NS)

        for j in range(min(G, n_chunks)):
            gather(j, j % NT)

        def body_spmem(i, carry):
            @pl.when(i - NS >= 0)
            def _():
                sel(i - NS, wait_write)

            @pl.when(i - 1 >= 0)
            def _():
                sel(i - 1, wait_t2s)
                sel(i - 1, write)

            @pl.when(i + G < n_chunks)
            def _():
                sel(i + G, gather)

            sel(i, wait_gather)
            sel(i, t2s)

            return carry

        def body_direct(i, carry):
            @pl.when(i + G - NT >= 0)
            def _():
                sel(i + G - NT, wait_dwrite)

            @pl.when(i + G < n_chunks)
            def _():
                sel(i + G, gather)

            sel(i, wait_gather)
            sel(i, dwrite)

            return carry

        direct = lax.rem(sid, 2) == 0

        @pl.when(direct)
        def _():
            lax.fori_loop(0, n_chunks, body_direct, 0)
            sel(n_chunks - 1, wait_dwrite)

        @pl.when(jnp.logical_not(direct))
        def _():
            lax.fori_loop(0, n_chunks, body_spmem, 0)
            sel(n_chunks - 1, wait_t2s)
            sel(n_chunks - 1, write)
            for j in range(max(0, n_chunks - NS), n_chunks):
                sel(j, wait_write)

    return k(flat_idx, table)


def kernel(input_ids, word_embeddings):
    b, s = input_ids.shape
    d = word_embeddings.shape[1]
    flat_idx = input_ids.reshape(b * s).astype(jnp.int32)
    out = _gather_rows(flat_idx, word_embeddings)
    return out.reshape(b, s, d)


# speedup vs baseline: 1.0211x; 1.0211x over previous
import functools
import math

import jax
import jax.numpy as jnp
from jax import lax
from jax.experimental import pallas as pl
from jax.experimental.pallas import tpu as pltpu
from jax.experimental.pallas import tpu_sc as plsc

NUM_CORES = 2
NUM_SUBCORES = 16
NUM_WORKERS = NUM_CORES * NUM_SUBCORES

CHUNK = 16
NT = 3
NS = 3
LOOKAHEAD = 2


@functools.partial(jax.jit, static_argnames=())
def _gather_rows(flat_idx, table):
    n = flat_idx.shape[0]
    d = table.shape[1]
    n_per_w = n // NUM_WORKERS
    n_chunks = n_per_w // CHUNK

    mesh = plsc.VectorSubcoreMesh(core_axis_name="c", subcore_axis_name="s")
    G = LOOKAHEAD

    @functools.partial(
        pl.kernel,
        mesh=mesh,
        out_type=jax.ShapeDtypeStruct((n, d), jnp.float32),
        scratch_types=[
            pltpu.VMEM((n_per_w,), jnp.int32),
            *[pltpu.VMEM((CHUNK, d), jnp.float32) for _ in range(NT)],
            pltpu.VMEM_SHARED((NUM_SUBCORES, NS, CHUNK, d), jnp.float32),
            *[pltpu.SemaphoreType.DMA for _ in range(NT + 2 * NS)],
        ],
    )
    def k(idx_hbm, table_hbm, out_hbm, idx_v, *rest):
        tbufs = rest[:NT]
        shared = rest[NT]
        sems = rest[NT + 1 :]
        gsems = sems[:NT]
        tsems = sems[NT : NT + NS]
        wsems = sems[NT + NS :]

        sid = lax.axis_index("s")
        wid = sid * NUM_CORES + lax.axis_index("c")
        base = wid * n_per_w
        pltpu.sync_copy(idx_hbm.at[pl.ds(base, n_per_w)], idx_v)

        def gather(i, t, s=0):
            pltpu.async_copy(
                table_hbm.at[idx_v.at[pl.ds(i * CHUNK, CHUNK)]], tbufs[t], gsems[t]
            )

        def wait_gather(i, t, s=0):
            pltpu.make_async_copy(
                table_hbm.at[idx_v.at[pl.ds(i * CHUNK, CHUNK)]], tbufs[t], gsems[t]
            ).wait()

        def t2s(i, t, s):
            pltpu.async_copy(tbufs[t], shared.at[sid, s], tsems[s])

        def wait_t2s(i, t, s):
            pltpu.make_async_copy(tbufs[t], shared.at[sid, s], tsems[s]).wait()

        def write(i, t, s):
            pltpu.async_copy(
                shared.at[sid, s], out_hbm.at[pl.ds(base + i * CHUNK, CHUNK)], wsems[s]
            )

        def wait_write(i, t, s):
            pltpu.make_async_copy(
                shared.at[sid, s], out_hbm.at[pl.ds(base + i * CHUNK, CHUNK)], wsems[s]
            ).wait()

        LCM = math.lcm(NT, NS)

        def sel(i, fn):
            for bb in range(LCM):

                @pl.when(lax.rem(i, LCM) == bb)
                def _():
                    fn(i, bb % NT, bb % NS)

        for j in range(min(G, n_chunks)):
            gather(j, j % NT)

        def body(i, carry):
            @pl.when(i - NS >= 0)
            def _():
                sel(i - NS, wait_write)

            @pl.when(i - 1 >= 0)
            def _():
                sel(i - 1, wait_t2s)
                sel(i - 1, write)

            @pl.when(i + G < n_chunks)
            def _():
                sel(i + G, gather)

            sel(i, wait_gather)
            sel(i, t2s)

            return carry

        lax.fori_loop(0, n_chunks, body, 0)
        sel(n_chunks - 1, wait_t2s)
        sel(n_chunks - 1, write)
        for j in range(max(0, n_chunks - NS), n_chunks):
            sel(j, wait_write)

    return k(flat_idx, table)


def kernel(input_ids, word_embeddings):
    b, s = input_ids.shape
    d = word_embeddings.shape[1]
    flat_idx = input_ids.reshape(b * s).astype(jnp.int32)
    out = _gather_rows(flat_idx, word_embeddings)
    return out.reshape(b, s, d)
